# baseline (device time: 281631 ns/iter reference)
import jax
import jax.numpy as jnp
from jax import lax
from jax.experimental import pallas as pl
from jax.experimental.pallas import tpu as pltpu

N_DEV = 16
H = 16
DH = 128
DR = 32
S = 1024
D = 2048


def _neighbor_barrier(left, right):
    barrier = pltpu.get_barrier_semaphore()
    for nbr in (left, right):
        pl.semaphore_signal(
            barrier, inc=1, device_id=(nbr,),
            device_id_type=pl.DeviceIdType.MESH,
        )
    pl.semaphore_wait(barrier, 2)


def _rs_body(kvp_ref, out_ref, comm_ref, send_sems, recv_sems, credit_sem):
    my = lax.axis_index("i")
    left = (my - 1) % N_DEV
    right = (my + 1) % N_DEV
    _neighbor_barrier(left, right)

    comm_ref[0] = kvp_ref[(my - 1) % N_DEV]
    for s in range(N_DEV - 1):
        send_slot = s % 2
        recv_slot = (s + 1) % 2
        if s >= 2:
            pl.semaphore_wait(credit_sem, 1)
        rdma = pltpu.make_async_remote_copy(
            src_ref=comm_ref.at[send_slot],
            dst_ref=comm_ref.at[recv_slot],
            send_sem=send_sems.at[send_slot],
            recv_sem=recv_sems.at[recv_slot],
            device_id=(right,),
            device_id_type=pl.DeviceIdType.MESH,
        )
        rdma.start()
        rdma.wait()
        if 1 <= s <= N_DEV - 3:
            pl.semaphore_signal(
                credit_sem, inc=1, device_id=(left,),
                device_id_type=pl.DeviceIdType.MESH,
            )
        k = (my - s - 2) % N_DEV
        if s < N_DEV - 2:
            comm_ref[recv_slot] = comm_ref[recv_slot] + kvp_ref[k]
        else:
            out_ref[...] = comm_ref[recv_slot] + kvp_ref[k]


def _reduce_scatter(kvp):
    return pl.pallas_call(
        _rs_body,
        out_shape=jax.ShapeDtypeStruct((2, S, DH), jnp.bfloat16),
        in_specs=[pl.BlockSpec(memory_space=pltpu.VMEM)],
        out_specs=pl.BlockSpec(memory_space=pltpu.VMEM),
        scratch_shapes=[
            pltpu.VMEM((2, 2, S, DH), jnp.bfloat16),
            pltpu.SemaphoreType.DMA((2,)),
            pltpu.SemaphoreType.DMA((2,)),
            pltpu.SemaphoreType.REGULAR,
        ],
        compiler_params=pltpu.CompilerParams(collective_id=0),
    )(kvp)


def _ag_body(o_ref, out_ref, send_sems, recv_sems):
    my = lax.axis_index("i")
    left = (my - 1) % N_DEV
    right = (my + 1) % N_DEV
    _neighbor_barrier(left, right)

    out_ref[my] = o_ref[...]
    for s in range(N_DEV - 1):
        k_send = (my - s) % N_DEV
        k_recv = (my - s - 1) % N_DEV
        send = pltpu.make_async_remote_copy(
            src_ref=out_ref.at[k_send],
            dst_ref=out_ref.at[k_send],
            send_sem=send_sems.at[s],
            recv_sem=recv_sems.at[s],
            device_id=(right,),
            device_id_type=pl.DeviceIdType.MESH,
        )
        send.start()
        recv = pltpu.make_async_remote_copy(
            src_ref=out_ref.at[k_recv],
            dst_ref=out_ref.at[k_recv],
            send_sem=send_sems.at[s],
            recv_sem=recv_sems.at[s],
            device_id=(left,),
            device_id_type=pl.DeviceIdType.MESH,
        )
        recv.wait_recv()
        send.wait_send()


def _all_gather(o_h):
    return pl.pallas_call(
        _ag_body,
        out_shape=jax.ShapeDtypeStruct((N_DEV, S, DH), jnp.bfloat16),
        in_specs=[pl.BlockSpec(memory_space=pltpu.VMEM)],
        out_specs=pl.BlockSpec(memory_space=pltpu.VMEM),
        scratch_shapes=[
            pltpu.SemaphoreType.DMA((N_DEV - 1,)),
            pltpu.SemaphoreType.DMA((N_DEV - 1,)),
        ],
        compiler_params=pltpu.CompilerParams(collective_id=1),
    )(o_h)


def kernel(x, Wdkv, Wuk, Wuv, Wq, Wqr, Wkr, Wo):
    my = lax.axis_index("i")
    bf = jnp.bfloat16
    xf = x[0].astype(bf)
    c = jnp.dot(xf, Wdkv.astype(bf))
    Kp = jnp.dot(c, Wuk.astype(bf))
    Vp = jnp.dot(c, Wuv.astype(bf))
    kvp = jnp.stack([Kp, Vp], axis=0)
    kvp = kvp.reshape(2, S, H, DH).transpose(2, 0, 1, 3)
    kv = _reduce_scatter(kvp)
    K_h, V_h = kv[0], kv[1]

    Wq_h = lax.dynamic_slice(Wq, (0, my * DH), (D, DH)).astype(bf)
    Wqr_h = lax.dynamic_slice(Wqr, (0, my * DR), (D, DR)).astype(bf)
    Q_h = jnp.dot(xf, Wq_h)
    Qr_h = jnp.dot(xf, Wqr_h)
    Kr = jnp.dot(xf, Wkr.astype(bf))

    scale = (DH + DR) ** -0.5
    scores = (
        jnp.dot(Q_h, K_h.T, preferred_element_type=jnp.float32)
        + jnp.dot(Qr_h, Kr.T, preferred_element_type=jnp.float32)
    ) * scale
    m = jnp.max(scores, axis=-1, keepdims=True)
    p = jnp.exp(scores - m)
    p = p / jnp.sum(p, axis=-1, keepdims=True)
    O_h = jnp.dot(p.astype(bf), V_h)

    O_all = _all_gather(O_h)
    O_full = O_all.transpose(1, 0, 2).reshape(S, H * DH)
    out = jnp.dot(O_full, Wo.astype(bf), preferred_element_type=jnp.float32)
    return out[None]


# device time: 186706 ns/iter; 1.5084x vs baseline; 1.5084x over previous
import jax
import jax.numpy as jnp
from jax import lax
from jax.experimental import pallas as pl
from jax.experimental.pallas import tpu as pltpu

N_DEV = 16
H = 16
DH = 128
DR = 32
S = 1024
D = 2048


def _neighbor_barrier(left, right):
    barrier = pltpu.get_barrier_semaphore()
    for nbr in (left, right):
        pl.semaphore_signal(
            barrier, inc=1, device_id=(nbr,),
            device_id_type=pl.DeviceIdType.MESH,
        )
    pl.semaphore_wait(barrier, 2)


_CCW_STEPS = 8
_CW_STEPS = 7


def _rs_body(kvp_ref, out_ref, ccw_ref, cw_ref,
             ccw_send_sems, ccw_recv_sems, cw_send_sems, cw_recv_sems,
             ccw_credit, cw_credit):
    my = lax.axis_index("i")
    left = (my - 1) % N_DEV
    right = (my + 1) % N_DEV
    _neighbor_barrier(left, right)

    ccw_ref[0] = kvp_ref[(my - _CCW_STEPS) % N_DEV]
    cw_ref[0] = kvp_ref[(my + _CW_STEPS) % N_DEV]
    for s in range(_CCW_STEPS):
        send_slot = s % 2
        recv_slot = (s + 1) % 2
        if s >= 2:
            pl.semaphore_wait(ccw_credit, 1)
        if 2 <= s < _CW_STEPS:
            pl.semaphore_wait(cw_credit, 1)
        ccw = pltpu.make_async_remote_copy(
            src_ref=ccw_ref.at[send_slot],
            dst_ref=ccw_ref.at[recv_slot],
            send_sem=ccw_send_sems.at[send_slot],
            recv_sem=ccw_recv_sems.at[recv_slot],
            device_id=(left,),
            device_id_type=pl.DeviceIdType.MESH,
        )
        ccw.start()
        if s < _CW_STEPS:
            cw = pltpu.make_async_remote_copy(
                src_ref=cw_ref.at[send_slot],
                dst_ref=cw_ref.at[recv_slot],
                send_sem=cw_send_sems.at[send_slot],
                recv_sem=cw_recv_sems.at[recv_slot],
                device_id=(right,),
                device_id_type=pl.DeviceIdType.MESH,
            )
            cw.start()
        ccw.wait()
        if 1 <= s <= _CCW_STEPS - 2:
            pl.semaphore_signal(
                ccw_credit, inc=1, device_id=(right,),
                device_id_type=pl.DeviceIdType.MESH,
            )
        if s < _CW_STEPS:
            cw.wait()
            if 1 <= s <= _CW_STEPS - 2:
                pl.semaphore_signal(
                    cw_credit, inc=1, device_id=(left,),
                    device_id_type=pl.DeviceIdType.MESH,
                )
        rc_ccw = (my - _CCW_STEPS + 1 + s) % N_DEV
        rc_cw = (my + _CW_STEPS - 1 - s) % N_DEV
        if s < _CCW_STEPS - 1:
            ccw_ref[recv_slot] = ccw_ref[recv_slot] + kvp_ref[rc_ccw]
        if s < _CW_STEPS - 1:
            cw_ref[recv_slot] = cw_ref[recv_slot] + kvp_ref[rc_cw]

    out_ref[...] = (
        ccw_ref[_CCW_STEPS % 2]
        + cw_ref[_CW_STEPS % 2]
        + kvp_ref[my]
    )


def _reduce_scatter(kvp):
    return pl.pallas_call(
        _rs_body,
        out_shape=jax.ShapeDtypeStruct((2, S, DH), jnp.bfloat16),
        in_specs=[pl.BlockSpec(memory_space=pltpu.VMEM)],
        out_specs=pl.BlockSpec(memory_space=pltpu.VMEM),
        scratch_shapes=[
            pltpu.VMEM((2, 2, S, DH), jnp.bfloat16),
            pltpu.VMEM((2, 2, S, DH), jnp.bfloat16),
            pltpu.SemaphoreType.DMA((2,)),
            pltpu.SemaphoreType.DMA((2,)),
            pltpu.SemaphoreType.DMA((2,)),
            pltpu.SemaphoreType.DMA((2,)),
            pltpu.SemaphoreType.REGULAR,
            pltpu.SemaphoreType.REGULAR,
        ],
        compiler_params=pltpu.CompilerParams(collective_id=0),
    )(kvp)


def _ag_body(o_ref, out_ref,
             cw_send_sems, cw_recv_sems, ccw_send_sems, ccw_recv_sems):
    my = lax.axis_index("i")
    left = (my - 1) % N_DEV
    right = (my + 1) % N_DEV
    _neighbor_barrier(left, right)

    out_ref[my] = o_ref[...]
    for s in range(8):
        k_cw_send = (my - s) % N_DEV
        k_cw_recv = (my - s - 1) % N_DEV
        cw = pltpu.make_async_remote_copy(
            src_ref=out_ref.at[k_cw_send],
            dst_ref=out_ref.at[k_cw_send],
            send_sem=cw_send_sems.at[s],
            recv_sem=cw_recv_sems.at[s],
            device_id=(right,),
            device_id_type=pl.DeviceIdType.MESH,
        )
        cw.start()
        if s < 7:
            k_ccw_send = (my + s) % N_DEV
            k_ccw_recv = (my + s + 1) % N_DEV
            ccw = pltpu.make_async_remote_copy(
                src_ref=out_ref.at[k_ccw_send],
                dst_ref=out_ref.at[k_ccw_send],
                send_sem=ccw_send_sems.at[s],
                recv_sem=ccw_recv_sems.at[s],
                device_id=(left,),
                device_id_type=pl.DeviceIdType.MESH,
            )
            ccw.start()
        cw_recv = pltpu.make_async_remote_copy(
            src_ref=out_ref.at[k_cw_recv],
            dst_ref=out_ref.at[k_cw_recv],
            send_sem=cw_send_sems.at[s],
            recv_sem=cw_recv_sems.at[s],
            device_id=(left,),
            device_id_type=pl.DeviceIdType.MESH,
        )
        cw_recv.wait_recv()
        cw.wait_send()
        if s < 7:
            ccw_recv = pltpu.make_async_remote_copy(
                src_ref=out_ref.at[k_ccw_recv],
                dst_ref=out_ref.at[k_ccw_recv],
                send_sem=ccw_send_sems.at[s],
                recv_sem=ccw_recv_sems.at[s],
                device_id=(right,),
                device_id_type=pl.DeviceIdType.MESH,
            )
            ccw_recv.wait_recv()
            ccw.wait_send()


def _all_gather(o_h):
    return pl.pallas_call(
        _ag_body,
        out_shape=jax.ShapeDtypeStruct((N_DEV, S, DH), jnp.bfloat16),
        in_specs=[pl.BlockSpec(memory_space=pltpu.VMEM)],
        out_specs=pl.BlockSpec(memory_space=pltpu.VMEM),
        scratch_shapes=[
            pltpu.SemaphoreType.DMA((8,)),
            pltpu.SemaphoreType.DMA((8,)),
            pltpu.SemaphoreType.DMA((7,)),
            pltpu.SemaphoreType.DMA((7,)),
        ],
        compiler_params=pltpu.CompilerParams(collective_id=1),
    )(o_h)


def kernel(x, Wdkv, Wuk, Wuv, Wq, Wqr, Wkr, Wo):
    my = lax.axis_index("i")
    bf = jnp.bfloat16
    xf = x[0].astype(bf)
    c = jnp.dot(xf, Wdkv.astype(bf))
    Kp = jnp.dot(c, Wuk.astype(bf))
    Vp = jnp.dot(c, Wuv.astype(bf))
    kvp = jnp.stack([Kp, Vp], axis=0)
    kvp = kvp.reshape(2, S, H, DH).transpose(2, 0, 1, 3)
    kv = _reduce_scatter(kvp)
    K_h, V_h = kv[0], kv[1]

    Wq_h = lax.dynamic_slice(Wq, (0, my * DH), (D, DH)).astype(bf)
    Wqr_h = lax.dynamic_slice(Wqr, (0, my * DR), (D, DR)).astype(bf)
    Q_h = jnp.dot(xf, Wq_h)
    Qr_h = jnp.dot(xf, Wqr_h)
    Kr = jnp.dot(xf, Wkr.astype(bf))

    scale = (DH + DR) ** -0.5
    scores = (
        jnp.dot(Q_h, K_h.T, preferred_element_type=jnp.float32)
        + jnp.dot(Qr_h, Kr.T, preferred_element_type=jnp.float32)
    ) * scale
    m = jnp.max(scores, axis=-1, keepdims=True)
    p = jnp.exp(scores - m)
    p = p / jnp.sum(p, axis=-1, keepdims=True)
    O_h = jnp.dot(p.astype(bf), V_h)

    O_all = _all_gather(O_h)
    O_full = O_all.transpose(1, 0, 2).reshape(S, H * DH)
    out = jnp.dot(O_full, Wo.astype(bf), preferred_element_type=jnp.float32)
    return out[None]


# device time: 155123 ns/iter; 1.8155x vs baseline; 1.2036x over previous
import jax
import jax.numpy as jnp
from jax import lax
from jax.experimental import pallas as pl
from jax.experimental.pallas import tpu as pltpu

N_DEV = 16
H = 16
DH = 128
DR = 32
S = 1024
D = 2048


def _ring_pos(mesh):
    q = mesh % 4
    z = mesh // 4
    return jnp.where(
        q == 0, z,
        jnp.where(q == 1, 7 - z, jnp.where(q == 2, 8 + z, 15 - z)))


def _mesh_of(r):
    r = r % N_DEV
    q = r // 4
    z = jnp.where(
        q == 0, r,
        jnp.where(q == 1, 7 - r, jnp.where(q == 2, r - 8, 15 - r)))
    return 4 * z + q


def _neighbor_barrier(left, right):
    barrier = pltpu.get_barrier_semaphore()
    for nbr in (left, right):
        pl.semaphore_signal(
            barrier, inc=1, device_id=(nbr,),
            device_id_type=pl.DeviceIdType.MESH,
        )
    pl.semaphore_wait(barrier, 2)


_CCW_STEPS = 8
_CW_STEPS = 7


def _rs_body(kvp_ref, out_ref, ccw_ref, cw_ref,
             ccw_send_sems, ccw_recv_sems, cw_send_sems, cw_recv_sems,
             ccw_credit, cw_credit):
    my_mesh = lax.axis_index("i")
    my = _ring_pos(my_mesh)
    left = _mesh_of(my - 1)
    right = _mesh_of(my + 1)
    _neighbor_barrier(left, right)

    ccw_ref[0] = kvp_ref[_mesh_of(my - _CCW_STEPS)]
    cw_ref[0] = kvp_ref[_mesh_of(my + _CW_STEPS)]
    for s in range(_CCW_STEPS):
        send_slot = s % 2
        recv_slot = (s + 1) % 2
        if s >= 2:
            pl.semaphore_wait(ccw_credit, 1)
        if 2 <= s < _CW_STEPS:
            pl.semaphore_wait(cw_credit, 1)
        ccw = pltpu.make_async_remote_copy(
            src_ref=ccw_ref.at[send_slot],
            dst_ref=ccw_ref.at[recv_slot],
            send_sem=ccw_send_sems.at[send_slot],
            recv_sem=ccw_recv_sems.at[recv_slot],
            device_id=(left,),
            device_id_type=pl.DeviceIdType.MESH,
        )
        ccw.start()
        if s < _CW_STEPS:
            cw = pltpu.make_async_remote_copy(
                src_ref=cw_ref.at[send_slot],
                dst_ref=cw_ref.at[recv_slot],
                send_sem=cw_send_sems.at[send_slot],
                recv_sem=cw_recv_sems.at[recv_slot],
                device_id=(right,),
                device_id_type=pl.DeviceIdType.MESH,
            )
            cw.start()
        ccw.wait()
        if 1 <= s <= _CCW_STEPS - 2:
            pl.semaphore_signal(
                ccw_credit, inc=1, device_id=(right,),
                device_id_type=pl.DeviceIdType.MESH,
            )
        if s < _CW_STEPS:
            cw.wait()
            if 1 <= s <= _CW_STEPS - 2:
                pl.semaphore_signal(
                    cw_credit, inc=1, device_id=(left,),
                    device_id_type=pl.DeviceIdType.MESH,
                )
        rc_ccw = _mesh_of(my - _CCW_STEPS + 1 + s)
        rc_cw = _mesh_of(my + _CW_STEPS - 1 - s)
        if s < _CCW_STEPS - 1:
            ccw_ref[recv_slot] = ccw_ref[recv_slot] + kvp_ref[rc_ccw]
        if s < _CW_STEPS - 1:
            cw_ref[recv_slot] = cw_ref[recv_slot] + kvp_ref[rc_cw]

    out_ref[...] = (
        ccw_ref[_CCW_STEPS % 2]
        + cw_ref[_CW_STEPS % 2]
        + kvp_ref[my_mesh]
    )


def _reduce_scatter(kvp):
    return pl.pallas_call(
        _rs_body,
        out_shape=jax.ShapeDtypeStruct((2, S, DH), jnp.bfloat16),
        in_specs=[pl.BlockSpec(memory_space=pltpu.VMEM)],
        out_specs=pl.BlockSpec(memory_space=pltpu.VMEM),
        scratch_shapes=[
            pltpu.VMEM((2, 2, S, DH), jnp.bfloat16),
            pltpu.VMEM((2, 2, S, DH), jnp.bfloat16),
            pltpu.SemaphoreType.DMA((2,)),
            pltpu.SemaphoreType.DMA((2,)),
            pltpu.SemaphoreType.DMA((2,)),
            pltpu.SemaphoreType.DMA((2,)),
            pltpu.SemaphoreType.REGULAR,
            pltpu.SemaphoreType.REGULAR,
        ],
        compiler_params=pltpu.CompilerParams(collective_id=0),
    )(kvp)


def _ag_body(o_ref, out_ref,
             cw_send_sems, cw_recv_sems, ccw_send_sems, ccw_recv_sems):
    my_mesh = lax.axis_index("i")
    my = _ring_pos(my_mesh)
    left = _mesh_of(my - 1)
    right = _mesh_of(my + 1)
    _neighbor_barrier(left, right)

    out_ref[my_mesh] = o_ref[...]
    for s in range(8):
        k_cw_send = _mesh_of(my - s)
        k_cw_recv = _mesh_of(my - s - 1)
        cw = pltpu.make_async_remote_copy(
            src_ref=out_ref.at[k_cw_send],
            dst_ref=out_ref.at[k_cw_send],
            send_sem=cw_send_sems.at[s],
            recv_sem=cw_recv_sems.at[s],
            device_id=(right,),
            device_id_type=pl.DeviceIdType.MESH,
        )
        cw.start()
        if s < 7:
            k_ccw_send = _mesh_of(my + s)
            k_ccw_recv = _mesh_of(my + s + 1)
            ccw = pltpu.make_async_remote_copy(
                src_ref=out_ref.at[k_ccw_send],
                dst_ref=out_ref.at[k_ccw_send],
                send_sem=ccw_send_sems.at[s],
                recv_sem=ccw_recv_sems.at[s],
                device_id=(left,),
                device_id_type=pl.DeviceIdType.MESH,
            )
            ccw.start()
        cw_recv = pltpu.make_async_remote_copy(
            src_ref=out_ref.at[k_cw_recv],
            dst_ref=out_ref.at[k_cw_recv],
            send_sem=cw_send_sems.at[s],
            recv_sem=cw_recv_sems.at[s],
            device_id=(left,),
            device_id_type=pl.DeviceIdType.MESH,
        )
        cw_recv.wait_recv()
        cw.wait_send()
        if s < 7:
            ccw_recv = pltpu.make_async_remote_copy(
                src_ref=out_ref.at[k_ccw_recv],
                dst_ref=out_ref.at[k_ccw_recv],
                send_sem=ccw_send_sems.at[s],
                recv_sem=ccw_recv_sems.at[s],
                device_id=(right,),
                device_id_type=pl.DeviceIdType.MESH,
            )
            ccw_recv.wait_recv()
            ccw.wait_send()


def _all_gather(o_h):
    return pl.pallas_call(
        _ag_body,
        out_shape=jax.ShapeDtypeStruct((N_DEV, S, DH), jnp.bfloat16),
        in_specs=[pl.BlockSpec(memory_space=pltpu.VMEM)],
        out_specs=pl.BlockSpec(memory_space=pltpu.VMEM),
        scratch_shapes=[
            pltpu.SemaphoreType.DMA((8,)),
            pltpu.SemaphoreType.DMA((8,)),
            pltpu.SemaphoreType.DMA((7,)),
            pltpu.SemaphoreType.DMA((7,)),
        ],
        compiler_params=pltpu.CompilerParams(collective_id=1),
    )(o_h)


def kernel(x, Wdkv, Wuk, Wuv, Wq, Wqr, Wkr, Wo):
    my = lax.axis_index("i")
    bf = jnp.bfloat16
    xf = x[0].astype(bf)
    c = jnp.dot(xf, Wdkv.astype(bf))
    Kp = jnp.dot(c, Wuk.astype(bf))
    Vp = jnp.dot(c, Wuv.astype(bf))
    kvp = jnp.stack([Kp, Vp], axis=0)
    kvp = kvp.reshape(2, S, H, DH).transpose(2, 0, 1, 3)
    kv = _reduce_scatter(kvp)
    K_h, V_h = kv[0], kv[1]

    Wq_h = lax.dynamic_slice(Wq, (0, my * DH), (D, DH)).astype(bf)
    Wqr_h = lax.dynamic_slice(Wqr, (0, my * DR), (D, DR)).astype(bf)
    Q_h = jnp.dot(xf, Wq_h)
    Qr_h = jnp.dot(xf, Wqr_h)
    Kr = jnp.dot(xf, Wkr.astype(bf))

    scale = (DH + DR) ** -0.5
    scores = (
        jnp.dot(Q_h, K_h.T, preferred_element_type=jnp.float32)
        + jnp.dot(Qr_h, Kr.T, preferred_element_type=jnp.float32)
    ) * scale
    m = jnp.max(scores, axis=-1, keepdims=True)
    p = jnp.exp(scores - m)
    p = p / jnp.sum(p, axis=-1, keepdims=True)
    O_h = jnp.dot(p.astype(bf), V_h)

    O_all = _all_gather(O_h)
    O_full = O_all.transpose(1, 0, 2).reshape(S, H * DH)
    out = jnp.dot(O_full, Wo.astype(bf), preferred_element_type=jnp.float32)
    return out[None]


# device time: 142042 ns/iter; 1.9827x vs baseline; 1.0921x over previous
import jax
import jax.numpy as jnp
from jax import lax
from jax.experimental import pallas as pl
from jax.experimental.pallas import tpu as pltpu

N_DEV = 16
H = 16
DH = 128
DR = 32
S = 1024
D = 2048


def _ring_pos(mesh):
    q = mesh % 4
    z = mesh // 4
    return jnp.where(
        q == 0, z,
        jnp.where(q == 1, 7 - z, jnp.where(q == 2, 8 + z, 15 - z)))


def _mesh_of(r):
    r = r % N_DEV
    q = r // 4
    z = jnp.where(
        q == 0, r,
        jnp.where(q == 1, 7 - r, jnp.where(q == 2, r - 8, 15 - r)))
    return 4 * z + q


def _neighbor_barrier(left, right):
    barrier = pltpu.get_barrier_semaphore()
    for nbr in (left, right):
        pl.semaphore_signal(
            barrier, inc=1, device_id=(nbr,),
            device_id_type=pl.DeviceIdType.MESH,
        )
    pl.semaphore_wait(barrier, 2)


_CCW_STEPS = 8
_CW_STEPS = 7


def _rs_body(kvp_ref, out_ref, ccw_ref, cw_ref,
             ccw_send_sems, ccw_recv_sems, cw_send_sems, cw_recv_sems,
             ccw_credit, cw_credit):
    my_mesh = lax.axis_index("i")
    my = _ring_pos(my_mesh)
    left = _mesh_of(my - 1)
    right = _mesh_of(my + 1)
    _neighbor_barrier(left, right)

    ccw_ref[0] = kvp_ref[_mesh_of(my - _CCW_STEPS)]
    cw_ref[0] = kvp_ref[_mesh_of(my + _CW_STEPS)]
    for s in range(_CCW_STEPS):
        send_slot = s % 2
        recv_slot = (s + 1) % 2
        if s >= 2:
            pl.semaphore_wait(ccw_credit, 1)
        if 2 <= s < _CW_STEPS:
            pl.semaphore_wait(cw_credit, 1)
        ccw = pltpu.make_async_remote_copy(
            src_ref=ccw_ref.at[send_slot],
            dst_ref=ccw_ref.at[recv_slot],
            send_sem=ccw_send_sems.at[send_slot],
            recv_sem=ccw_recv_sems.at[recv_slot],
            device_id=(left,),
            device_id_type=pl.DeviceIdType.MESH,
        )
        ccw.start()
        if s < _CW_STEPS:
            cw = pltpu.make_async_remote_copy(
                src_ref=cw_ref.at[send_slot],
                dst_ref=cw_ref.at[recv_slot],
                send_sem=cw_send_sems.at[send_slot],
                recv_sem=cw_recv_sems.at[recv_slot],
                device_id=(right,),
                device_id_type=pl.DeviceIdType.MESH,
            )
            cw.start()
        ccw.wait()
        if 1 <= s <= _CCW_STEPS - 2:
            pl.semaphore_signal(
                ccw_credit, inc=1, device_id=(right,),
                device_id_type=pl.DeviceIdType.MESH,
            )
        if s < _CW_STEPS:
            cw.wait()
            if 1 <= s <= _CW_STEPS - 2:
                pl.semaphore_signal(
                    cw_credit, inc=1, device_id=(left,),
                    device_id_type=pl.DeviceIdType.MESH,
                )
        rc_ccw = _mesh_of(my - _CCW_STEPS + 1 + s)
        rc_cw = _mesh_of(my + _CW_STEPS - 1 - s)
        if s < _CCW_STEPS - 1:
            ccw_ref[recv_slot] = ccw_ref[recv_slot] + kvp_ref[rc_ccw]
        if s < _CW_STEPS - 1:
            cw_ref[recv_slot] = cw_ref[recv_slot] + kvp_ref[rc_cw]

    out_ref[...] = (
        ccw_ref[_CCW_STEPS % 2]
        + cw_ref[_CW_STEPS % 2]
        + kvp_ref[my_mesh]
    )


def _reduce_scatter(kvp):
    return pl.pallas_call(
        _rs_body,
        out_shape=jax.ShapeDtypeStruct((2, S, DH), jnp.bfloat16),
        in_specs=[pl.BlockSpec(memory_space=pltpu.VMEM)],
        out_specs=pl.BlockSpec(memory_space=pltpu.VMEM),
        scratch_shapes=[
            pltpu.VMEM((2, 2, S, DH), jnp.bfloat16),
            pltpu.VMEM((2, 2, S, DH), jnp.bfloat16),
            pltpu.SemaphoreType.DMA((2,)),
            pltpu.SemaphoreType.DMA((2,)),
            pltpu.SemaphoreType.DMA((2,)),
            pltpu.SemaphoreType.DMA((2,)),
            pltpu.SemaphoreType.REGULAR,
            pltpu.SemaphoreType.REGULAR,
        ],
        compiler_params=pltpu.CompilerParams(collective_id=0),
    )(kvp)


def _ag_body(o_ref, wo_ref, out_ref, slots,
             cw_send_sems, cw_recv_sems, ccw_send_sems, ccw_recv_sems):
    my_mesh = lax.axis_index("i")
    my = _ring_pos(my_mesh)
    left = _mesh_of(my - 1)
    right = _mesh_of(my + 1)
    _neighbor_barrier(left, right)

    slots[my_mesh] = o_ref[...]

    def head_out(k):
        return jnp.dot(
            slots[k], wo_ref[pl.ds(k * DH, DH), :],
            preferred_element_type=jnp.float32,
        )

    for s in range(8):
        k_cw_send = _mesh_of(my - s)
        k_cw_recv = _mesh_of(my - s - 1)
        cw = pltpu.make_async_remote_copy(
            src_ref=slots.at[k_cw_send],
            dst_ref=slots.at[k_cw_send],
            send_sem=cw_send_sems.at[s],
            recv_sem=cw_recv_sems.at[s],
            device_id=(right,),
            device_id_type=pl.DeviceIdType.MESH,
        )
        cw.start()
        if s < 7:
            k_ccw_send = _mesh_of(my + s)
            k_ccw_recv = _mesh_of(my + s + 1)
            ccw = pltpu.make_async_remote_copy(
                src_ref=slots.at[k_ccw_send],
                dst_ref=slots.at[k_ccw_send],
                send_sem=ccw_send_sems.at[s],
                recv_sem=ccw_recv_sems.at[s],
                device_id=(left,),
                device_id_type=pl.DeviceIdType.MESH,
            )
            ccw.start()
        if s == 0:
            out_ref[...] = head_out(my_mesh)
        else:
            out_ref[...] = out_ref[...] + head_out(_mesh_of(my - s))
            out_ref[...] = out_ref[...] + head_out(_mesh_of(my + s))
        cw_recv = pltpu.make_async_remote_copy(
            src_ref=slots.at[k_cw_recv],
            dst_ref=slots.at[k_cw_recv],
            send_sem=cw_send_sems.at[s],
            recv_sem=cw_recv_sems.at[s],
            device_id=(left,),
            device_id_type=pl.DeviceIdType.MESH,
        )
        cw_recv.wait_recv()
        cw.wait_send()
        if s < 7:
            ccw_recv = pltpu.make_async_remote_copy(
                src_ref=slots.at[k_ccw_recv],
                dst_ref=slots.at[k_ccw_recv],
                send_sem=ccw_send_sems.at[s],
                recv_sem=ccw_recv_sems.at[s],
                device_id=(right,),
                device_id_type=pl.DeviceIdType.MESH,
            )
            ccw_recv.wait_recv()
            ccw.wait_send()
    out_ref[...] = out_ref[...] + head_out(_mesh_of(my - 8))


def _all_gather_matmul(o_h, wo):
    return pl.pallas_call(
        _ag_body,
        out_shape=jax.ShapeDtypeStruct((S, D), jnp.float32),
        in_specs=[
            pl.BlockSpec(memory_space=pltpu.VMEM),
            pl.BlockSpec(memory_space=pltpu.VMEM),
        ],
        out_specs=pl.BlockSpec(memory_space=pltpu.VMEM),
        scratch_shapes=[
            pltpu.VMEM((N_DEV, S, DH), jnp.bfloat16),
            pltpu.SemaphoreType.DMA((8,)),
            pltpu.SemaphoreType.DMA((8,)),
            pltpu.SemaphoreType.DMA((7,)),
            pltpu.SemaphoreType.DMA((7,)),
        ],
        compiler_params=pltpu.CompilerParams(collective_id=1),
    )(o_h, wo)


def kernel(x, Wdkv, Wuk, Wuv, Wq, Wqr, Wkr, Wo):
    my = lax.axis_index("i")
    bf = jnp.bfloat16
    xf = x[0].astype(bf)
    c = jnp.dot(xf, Wdkv.astype(bf))
    Kp = jnp.dot(c, Wuk.astype(bf))
    Vp = jnp.dot(c, Wuv.astype(bf))
    kvp = jnp.stack([Kp, Vp], axis=0)
    kvp = kvp.reshape(2, S, H, DH).transpose(2, 0, 1, 3)
    kv = _reduce_scatter(kvp)
    K_h, V_h = kv[0], kv[1]

    Wq_h = lax.dynamic_slice(Wq, (0, my * DH), (D, DH)).astype(bf)
    Wqr_h = lax.dynamic_slice(Wqr, (0, my * DR), (D, DR)).astype(bf)
    Q_h = jnp.dot(xf, Wq_h)
    Qr_h = jnp.dot(xf, Wqr_h)
    Kr = jnp.dot(xf, Wkr.astype(bf))

    scale = (DH + DR) ** -0.5
    scores = (
        jnp.dot(Q_h, K_h.T, preferred_element_type=jnp.float32)
        + jnp.dot(Qr_h, Kr.T, preferred_element_type=jnp.float32)
    ) * scale
    m = jnp.max(scores, axis=-1, keepdims=True)
    p = jnp.exp(scores - m)
    p = p / jnp.sum(p, axis=-1, keepdims=True)
    O_h = jnp.dot(p.astype(bf), V_h)

    out = _all_gather_matmul(O_h, Wo.astype(bf))
    return out[None]


# device time: 116412 ns/iter; 2.4193x vs baseline; 1.2202x over previous
import jax
import jax.numpy as jnp
from jax import lax
from jax.experimental import pallas as pl
from jax.experimental.pallas import tpu as pltpu

N_DEV = 16
H = 16
DH = 128
DR = 32
DC = 128
S = 1024
D = 2048


def _ring_pos(mesh):
    q = mesh % 4
    z = mesh // 4
    return jnp.where(
        q == 0, z,
        jnp.where(q == 1, 7 - z, jnp.where(q == 2, 8 + z, 15 - z)))


def _mesh_of(r):
    r = r % N_DEV
    q = r // 4
    z = jnp.where(
        q == 0, r,
        jnp.where(q == 1, 7 - r, jnp.where(q == 2, r - 8, 15 - r)))
    return 4 * z + q


def _neighbor_barrier(left, right):
    barrier = pltpu.get_barrier_semaphore()
    for nbr in (left, right):
        pl.semaphore_signal(
            barrier, inc=1, device_id=(nbr,),
            device_id_type=pl.DeviceIdType.MESH,
        )
    pl.semaphore_wait(barrier, 2)


def _prep_body(c_ref, wuk_ref, wuv_ref, xf_ref, wq_ref, wqr_ref, wkr_ref,
               kv_ref, q_ref, qr_ref, kr_ref,
               c_all, wstage, wrecv, kvacc,
               a2a_send_sems, a2a_recv_sems,
               cw_send_sems, cw_recv_sems, ccw_send_sems, ccw_recv_sems):
    my_mesh = lax.axis_index("i")
    my = _ring_pos(my_mesh)
    left = _mesh_of(my - 1)
    right = _mesh_of(my + 1)

    barrier = pltpu.get_barrier_semaphore()
    for off in range(1, N_DEV):
        pl.semaphore_signal(
            barrier, inc=1, device_id=((my_mesh + off) % N_DEV,),
            device_id_type=pl.DeviceIdType.MESH,
        )
    pl.semaphore_wait(barrier, N_DEV - 1)

    for d in range(N_DEV):
        wstage[d, 0] = wuk_ref[:, d * DH:(d + 1) * DH]
        wstage[d, 1] = wuv_ref[:, d * DH:(d + 1) * DH]
    wrecv[my_mesh] = wstage[my_mesh]
    a2a = []
    for off in range(1, N_DEV):
        dst = (my_mesh + off) % N_DEV
        r = pltpu.make_async_remote_copy(
            src_ref=wstage.at[dst],
            dst_ref=wrecv.at[my_mesh],
            send_sem=a2a_send_sems.at[dst],
            recv_sem=a2a_recv_sems.at[my_mesh],
            device_id=(dst,),
            device_id_type=pl.DeviceIdType.MESH,
        )
        r.start()
        a2a.append(r)

    c_all[my_mesh] = c_ref[...]
    kvacc[0] = jnp.dot(c_ref[...], wrecv[my_mesh, 0],
                       preferred_element_type=jnp.float32)
    kvacc[1] = jnp.dot(c_ref[...], wrecv[my_mesh, 1],
                       preferred_element_type=jnp.float32)

    def absorb(k):
        w_arrived = pltpu.make_async_remote_copy(
            src_ref=wstage.at[k],
            dst_ref=wrecv.at[k],
            send_sem=a2a_send_sems.at[k],
            recv_sem=a2a_recv_sems.at[k],
            device_id=(k,),
            device_id_type=pl.DeviceIdType.MESH,
        )
        w_arrived.wait_recv()
        kvacc[0] = kvacc[0] + jnp.dot(
            c_all[k], wrecv[k, 0], preferred_element_type=jnp.float32)
        kvacc[1] = kvacc[1] + jnp.dot(
            c_all[k], wrecv[k, 1], preferred_element_type=jnp.float32)

    for s in range(8):
        k_cw_send = _mesh_of(my - s)
        k_cw_recv = _mesh_of(my - s - 1)
        cw = pltpu.make_async_remote_copy(
            src_ref=c_all.at[k_cw_send],
            dst_ref=c_all.at[k_cw_send],
            send_sem=cw_send_sems.at[s],
            recv_sem=cw_recv_sems.at[s],
            device_id=(right,),
            device_id_type=pl.DeviceIdType.MESH,
        )
        cw.start()
        if s < 7:
            k_ccw_send = _mesh_of(my + s)
            k_ccw_recv = _mesh_of(my + s + 1)
            ccw = pltpu.make_async_remote_copy(
                src_ref=c_all.at[k_ccw_send],
                dst_ref=c_all.at[k_ccw_send],
                send_sem=ccw_send_sems.at[s],
                recv_sem=ccw_recv_sems.at[s],
                device_id=(left,),
                device_id_type=pl.DeviceIdType.MESH,
            )
            ccw.start()
        if s >= 1:
            absorb(_mesh_of(my - s))
            absorb(_mesh_of(my + s))
        if s == 0:
            q_ref[...] = jnp.dot(
                xf_ref[...], wq_ref[...],
                preferred_element_type=jnp.float32).astype(jnp.bfloat16)
        if s == 1:
            qr_ref[...] = jnp.dot(
                xf_ref[...], wqr_ref[...],
                preferred_element_type=jnp.float32).astype(jnp.bfloat16)
            kr_ref[...] = jnp.dot(
                xf_ref[...], wkr_ref[...],
                preferred_element_type=jnp.float32).astype(jnp.bfloat16)
        cw_recv = pltpu.make_async_remote_copy(
            src_ref=c_all.at[k_cw_recv],
            dst_ref=c_all.at[k_cw_recv],
            send_sem=cw_send_sems.at[s],
            recv_sem=cw_recv_sems.at[s],
            device_id=(left,),
            device_id_type=pl.DeviceIdType.MESH,
        )
        cw_recv.wait_recv()
        cw.wait_send()
        if s < 7:
            ccw_recv = pltpu.make_async_remote_copy(
                src_ref=c_all.at[k_ccw_recv],
                dst_ref=c_all.at[k_ccw_recv],
                send_sem=ccw_send_sems.at[s],
                recv_sem=ccw_recv_sems.at[s],
                device_id=(right,),
                device_id_type=pl.DeviceIdType.MESH,
            )
            ccw_recv.wait_recv()
            ccw.wait_send()
    absorb(_mesh_of(my - 8))
    kv_ref[0] = kvacc[0].astype(jnp.bfloat16)
    kv_ref[1] = kvacc[1].astype(jnp.bfloat16)
    for r in a2a:
        r.wait_send()


def _prepare(c, wuk, wuv, xf, wq_h, wqr_h, wkr):
    return pl.pallas_call(
        _prep_body,
        out_shape=(
            jax.ShapeDtypeStruct((2, S, DH), jnp.bfloat16),
            jax.ShapeDtypeStruct((S, DH), jnp.bfloat16),
            jax.ShapeDtypeStruct((S, DR), jnp.bfloat16),
            jax.ShapeDtypeStruct((S, DR), jnp.bfloat16),
        ),
        in_specs=[pl.BlockSpec(memory_space=pltpu.VMEM)] * 7,
        out_specs=(pl.BlockSpec(memory_space=pltpu.VMEM),) * 4,
        scratch_shapes=[
            pltpu.VMEM((N_DEV, S, DC), jnp.bfloat16),
            pltpu.VMEM((N_DEV, 2, DC, DH), jnp.bfloat16),
            pltpu.VMEM((N_DEV, 2, DC, DH), jnp.bfloat16),
            pltpu.VMEM((2, S, DH), jnp.float32),
            pltpu.SemaphoreType.DMA((N_DEV,)),
            pltpu.SemaphoreType.DMA((N_DEV,)),
            pltpu.SemaphoreType.DMA((8,)),
            pltpu.SemaphoreType.DMA((8,)),
            pltpu.SemaphoreType.DMA((7,)),
            pltpu.SemaphoreType.DMA((7,)),
        ],
        compiler_params=pltpu.CompilerParams(collective_id=0),
    )(c, wuk, wuv, xf, wq_h, wqr_h, wkr)


def _ag_body(o_ref, wo_ref, out_ref, slots,
             cw_send_sems, cw_recv_sems, ccw_send_sems, ccw_recv_sems):
    my_mesh = lax.axis_index("i")
    my = _ring_pos(my_mesh)
    left = _mesh_of(my - 1)
    right = _mesh_of(my + 1)
    _neighbor_barrier(left, right)

    slots[my_mesh] = o_ref[...]

    def head_out(k):
        return jnp.dot(
            slots[k], wo_ref[pl.ds(k * DH, DH), :],
            preferred_element_type=jnp.float32,
        )

    for s in range(8):
        k_cw_send = _mesh_of(my - s)
        k_cw_recv = _mesh_of(my - s - 1)
        cw = pltpu.make_async_remote_copy(
            src_ref=slots.at[k_cw_send],
            dst_ref=slots.at[k_cw_send],
            send_sem=cw_send_sems.at[s],
            recv_sem=cw_recv_sems.at[s],
            device_id=(right,),
            device_id_type=pl.DeviceIdType.MESH,
        )
        cw.start()
        if s < 7:
            k_ccw_send = _mesh_of(my + s)
            k_ccw_recv = _mesh_of(my + s + 1)
            ccw = pltpu.make_async_remote_copy(
                src_ref=slots.at[k_ccw_send],
                dst_ref=slots.at[k_ccw_send],
                send_sem=ccw_send_sems.at[s],
                recv_sem=ccw_recv_sems.at[s],
                device_id=(left,),
                device_id_type=pl.DeviceIdType.MESH,
            )
            ccw.start()
        if s == 0:
            out_ref[...] = head_out(my_mesh)
        else:
            out_ref[...] = out_ref[...] + head_out(_mesh_of(my - s))
            out_ref[...] = out_ref[...] + head_out(_mesh_of(my + s))
        cw_recv = pltpu.make_async_remote_copy(
            src_ref=slots.at[k_cw_recv],
            dst_ref=slots.at[k_cw_recv],
            send_sem=cw_send_sems.at[s],
            recv_sem=cw_recv_sems.at[s],
            device_id=(left,),
            device_id_type=pl.DeviceIdType.MESH,
        )
        cw_recv.wait_recv()
        cw.wait_send()
        if s < 7:
            ccw_recv = pltpu.make_async_remote_copy(
                src_ref=slots.at[k_ccw_recv],
                dst_ref=slots.at[k_ccw_recv],
                send_sem=ccw_send_sems.at[s],
                recv_sem=ccw_recv_sems.at[s],
                device_id=(right,),
                device_id_type=pl.DeviceIdType.MESH,
            )
            ccw_recv.wait_recv()
            ccw.wait_send()
    out_ref[...] = out_ref[...] + head_out(_mesh_of(my - 8))


def _all_gather_matmul(o_h, wo):
    return pl.pallas_call(
        _ag_body,
        out_shape=jax.ShapeDtypeStruct((S, D), jnp.float32),
        in_specs=[
            pl.BlockSpec(memory_space=pltpu.VMEM),
            pl.BlockSpec(memory_space=pltpu.VMEM),
        ],
        out_specs=pl.BlockSpec(memory_space=pltpu.VMEM),
        scratch_shapes=[
            pltpu.VMEM((N_DEV, S, DH), jnp.bfloat16),
            pltpu.SemaphoreType.DMA((8,)),
            pltpu.SemaphoreType.DMA((8,)),
            pltpu.SemaphoreType.DMA((7,)),
            pltpu.SemaphoreType.DMA((7,)),
        ],
        compiler_params=pltpu.CompilerParams(collective_id=1),
    )(o_h, wo)


def kernel(x, Wdkv, Wuk, Wuv, Wq, Wqr, Wkr, Wo):
    my = lax.axis_index("i")
    bf = jnp.bfloat16
    xf = x[0].astype(bf)
    c = jnp.dot(xf, Wdkv.astype(bf))
    Wq_h = lax.dynamic_slice(Wq, (0, my * DH), (D, DH)).astype(bf)
    Wqr_h = lax.dynamic_slice(Wqr, (0, my * DR), (D, DR)).astype(bf)
    kv, Q_h, Qr_h, Kr = _prepare(
        c, Wuk.astype(bf), Wuv.astype(bf), xf, Wq_h, Wqr_h, Wkr.astype(bf))
    K_h, V_h = kv[0], kv[1]

    scale = (DH + DR) ** -0.5
    scores = (
        jnp.dot(Q_h, K_h.T, preferred_element_type=jnp.float32)
        + jnp.dot(Qr_h, Kr.T, preferred_element_type=jnp.float32)
    ) * scale
    m = jnp.max(scores, axis=-1, keepdims=True)
    p = jnp.exp(scores - m)
    p = p / jnp.sum(p, axis=-1, keepdims=True)
    O_h = jnp.dot(p.astype(bf), V_h)

    out = _all_gather_matmul(O_h, Wo.astype(bf))
    return out[None]
